# baseline (speedup 1.0000x reference)
import functools

import jax
import jax.numpy as jnp
from jax import lax
from jax.experimental import pallas as pl
from jax.experimental.pallas import tpu as pltpu
from jax.experimental.pallas import tpu_sc as plsc

_NB_KNOTS = 128
_X_MIN = -3.0
_X_MAX = 3.0
_EPS = 1e-06

_NC = 2
_NS = 16
_NW = _NC * _NS
_L = 16

_ROWS = 2048
_COLS = 4096
_RPW = _ROWS // _NW
_CR = 8
_CC = _COLS
_NBUF = 3
_NBLK = _RPW // _CR


def _build_tables(theta_ref, y_ref, dy_ref):
    lane = lax.iota(jnp.int32, _L)
    zero = jnp.zeros((_L,), jnp.float32)
    carry = zero
    for k in range(_NB_KNOTS // _L):
        v = theta_ref[pl.ds(k * _L, _L)]
        d = jnp.exp(v) + jnp.float32(_EPS)
        if k == 0:
            d = jnp.where(lane == 0, v, d)
        c = d
        for s in (1, 2, 4, 8):
            y_ref[pl.ds(k * _L, _L)] = c
            shifted = plsc.load_gather(
                y_ref, [jnp.maximum(lane - s, 0) + k * _L])
            c = c + jnp.where(lane >= s, shifted, zero)
        c = c + carry
        y_ref[pl.ds(k * _L, _L)] = c
        carry = plsc.load_gather(
            y_ref, [jnp.full((_L,), k * _L + _L - 1, jnp.int32)]
        )
    for k in range(_NB_KNOTS // _L):
        idx = lane + k * _L
        yl = plsc.load_gather(y_ref, [idx])
        yr = plsc.load_gather(y_ref, [jnp.minimum(idx + 1, _NB_KNOTS - 1)])
        dy_ref[pl.ds(k * _L, _L)] = yr - yl


def _interp_block(inb, outb, y_ref, dy_ref):
    scale = jnp.float32((_NB_KNOTS - 1) / (_X_MAX - _X_MIN))
    for r in range(_CR):
        @plsc.parallel_loop(0, _CC, step=_L, unroll=4)
        def body(off):
            zv = inb[r, pl.ds(off, _L)]
            zn = (zv - jnp.float32(_X_MIN)) * scale
            znc = jnp.minimum(jnp.maximum(zn, jnp.float32(0.0)),
                              jnp.float32(_NB_KNOTS - 2))
            ii = znc.astype(jnp.int32)
            t = zn - ii.astype(jnp.float32)
            yl = plsc.load_gather(y_ref, [ii])
            dy = plsc.load_gather(dy_ref, [ii])
            outb[r, pl.ds(off, _L)] = yl + t * dy


@functools.partial(
    pl.kernel,
    mesh=plsc.VectorSubcoreMesh(core_axis_name="c", subcore_axis_name="s"),
    out_type=jax.ShapeDtypeStruct((_ROWS, _COLS), jnp.float32),
    compiler_params=pltpu.CompilerParams(needs_layout_passes=False),
    scratch_types=[
        pltpu.VMEM((_NB_KNOTS,), jnp.float32),
        pltpu.VMEM((_NB_KNOTS,), jnp.float32),
        pltpu.VMEM((_NB_KNOTS,), jnp.float32),
        pltpu.VMEM((_CR, _CC), jnp.float32),
        pltpu.VMEM((_CR, _CC), jnp.float32),
        pltpu.VMEM((_CR, _CC), jnp.float32),
        pltpu.SemaphoreType.DMA,
        pltpu.SemaphoreType.DMA,
        pltpu.SemaphoreType.DMA,
        pltpu.SemaphoreType.DMA,
        pltpu.SemaphoreType.DMA,
        pltpu.SemaphoreType.DMA,
    ],
)
def _spline_sc(z_hbm, theta_hbm, out_hbm,
               theta_v, y_v, dy_v,
               b0, b1, b2, si0, si1, si2, so0, so1, so2):
    wid = lax.axis_index("s") * _NC + lax.axis_index("c")
    row0 = wid * _RPW
    bufs = (b0, b1, b2)
    sis = (si0, si1, si2)
    sos = (so0, so1, so2)

    pltpu.sync_copy(theta_hbm, theta_v)
    _build_tables(theta_v, y_v, dy_v)

    def blk_in(c):
        return z_hbm.at[pl.ds(row0 + c * _CR, _CR), :]

    def blk_out(c):
        return out_hbm.at[pl.ds(row0 + c * _CR, _CR), :]

    pltpu.async_copy(blk_in(0), bufs[0], sis[0])
    for c in range(_NBLK):
        b = c % _NBUF
        bn = (c + 1) % _NBUF
        pltpu.make_async_copy(blk_in(c), bufs[b], sis[b]).wait()
        if c >= 2 and c + 1 < _NBLK:
            pltpu.make_async_copy(bufs[bn], blk_out(c - 2), sos[bn]).wait()
        if c + 1 < _NBLK:
            pltpu.async_copy(blk_in(c + 1), bufs[bn], sis[bn])
        _interp_block(bufs[b], bufs[b], y_v, dy_v)
        pltpu.async_copy(bufs[b], blk_out(c), sos[b])
    for c in (_NBLK - 3, _NBLK - 2, _NBLK - 1):
        b = c % _NBUF
        pltpu.make_async_copy(bufs[b], blk_out(c), sos[b]).wait()


def kernel(z, theta):
    return _spline_sc(z, theta)


# speedup vs baseline: 1.0497x; 1.0497x over previous
import functools

import jax
import jax.numpy as jnp
from jax import lax
from jax.experimental import pallas as pl
from jax.experimental.pallas import tpu as pltpu
from jax.experimental.pallas import tpu_sc as plsc

_NB_KNOTS = 128
_X_MIN = -3.0
_X_MAX = 3.0
_EPS = 1e-06
_SCALE = (_NB_KNOTS - 1) / (_X_MAX - _X_MIN)

_NC = 2
_NS = 16
_NW = _NC * _NS
_L = 16

_ROWS = 2048
_COLS = 4096
_SC_ROWS = 768
_TC_ROWS = _ROWS - _SC_ROWS
_TC_BR = 256

_RPW = _SC_ROWS // _NW
_CR = 8
_CC = _COLS // 2
_NG = _RPW // _CR


def _build_tables(theta_ref, y_ref, dy_ref):
    lane = lax.iota(jnp.int32, _L)
    zero = jnp.zeros((_L,), jnp.float32)
    carry = zero
    for k in range(_NB_KNOTS // _L):
        v = theta_ref[pl.ds(k * _L, _L)]
        d = jnp.exp(v) + jnp.float32(_EPS)
        if k == 0:
            d = jnp.where(lane == 0, v, d)
        c = d
        for s in (1, 2, 4, 8):
            y_ref[pl.ds(k * _L, _L)] = c
            shifted = plsc.load_gather(
                y_ref, [jnp.maximum(lane - s, 0) + k * _L])
            c = c + jnp.where(lane >= s, shifted, zero)
        c = c + carry
        y_ref[pl.ds(k * _L, _L)] = c
        carry = plsc.load_gather(
            y_ref, [jnp.full((_L,), k * _L + _L - 1, jnp.int32)]
        )
    for k in range(_NB_KNOTS // _L):
        idx = lane + k * _L
        yl = plsc.load_gather(y_ref, [idx])
        yr = plsc.load_gather(y_ref, [jnp.minimum(idx + 1, _NB_KNOTS - 1)])
        dy_ref[pl.ds(k * _L, _L)] = yr - yl


def _interp_block(inb, outb, y_ref, dy_ref):
    for r in range(_CR):
        @plsc.parallel_loop(0, _CC, step=_L, unroll=4)
        def body(off):
            zv = inb[r, pl.ds(off, _L)]
            zn = (zv - jnp.float32(_X_MIN)) * jnp.float32(_SCALE)
            znc = jnp.minimum(jnp.maximum(zn, jnp.float32(0.0)),
                              jnp.float32(_NB_KNOTS - 2))
            ii = znc.astype(jnp.int32)
            t = zn - ii.astype(jnp.float32)
            yl = plsc.load_gather(y_ref, [ii])
            dy = plsc.load_gather(dy_ref, [ii])
            outb[r, pl.ds(off, _L)] = yl + t * dy


@functools.partial(
    pl.kernel,
    mesh=plsc.VectorSubcoreMesh(core_axis_name="c", subcore_axis_name="s"),
    out_type=jax.ShapeDtypeStruct((_SC_ROWS, _COLS), jnp.float32),
    compiler_params=pltpu.CompilerParams(needs_layout_passes=False),
    scratch_types=[
        pltpu.VMEM((_NB_KNOTS,), jnp.float32),
        pltpu.VMEM((_NB_KNOTS,), jnp.float32),
        pltpu.VMEM((_NB_KNOTS,), jnp.float32),
        pltpu.VMEM((_CR, _CC), jnp.float32),
        pltpu.VMEM((_CR, _CC), jnp.float32),
        pltpu.VMEM((_CR, _CC), jnp.float32),
        pltpu.VMEM((_CR, _CC), jnp.float32),
        pltpu.SemaphoreType.DMA,
        pltpu.SemaphoreType.DMA,
        pltpu.SemaphoreType.DMA,
        pltpu.SemaphoreType.DMA,
    ],
)
def _spline_sc(z_hbm, theta_hbm, out_hbm,
               theta_v, y_v, dy_v, ib0, ib1, ob0, ob1, si0, si1, so0, so1):
    wid = lax.axis_index("s") * _NC + lax.axis_index("c")
    row0 = wid * _RPW

    pltpu.sync_copy(theta_hbm, theta_v)
    _build_tables(theta_v, y_v, dy_v)

    def in_slice(g, b):
        return z_hbm.at[pl.ds(row0 + g * _CR, _CR), pl.ds(b * _CC, _CC)]

    def out_slice(g, b):
        return out_hbm.at[pl.ds(row0 + g * _CR, _CR), pl.ds(b * _CC, _CC)]

    pltpu.async_copy(in_slice(0, 0), ib0, si0)
    pltpu.async_copy(in_slice(0, 1), ib1, si1)

    def group(g, carry):
        pairs = ((ib0, ob0, si0, so0), (ib1, ob1, si1, so1))
        for b, (inb, outb, si, so) in enumerate(pairs):
            pltpu.make_async_copy(in_slice(g, b), inb, si).wait()

            @pl.when(g > 0)
            def _():
                pltpu.make_async_copy(outb, out_slice(g, b), so).wait()

            _interp_block(inb, outb, y_v, dy_v)
            pltpu.async_copy(outb, out_slice(g, b), so)

            @pl.when(g + 1 < _NG)
            def _():
                pltpu.async_copy(in_slice(g + 1, b), inb, si)
        return carry

    lax.fori_loop(0, _NG, group, None)
    pltpu.make_async_copy(ob0, out_slice(_NG - 1, 0), so0).wait()
    pltpu.make_async_copy(ob1, out_slice(_NG - 1, 1), so1).wait()


def _spline_tc_body(z_ref, th_ref, o_ref):
    th = th_ref[...]
    col = lax.broadcasted_iota(jnp.int32, (1, _NB_KNOTS), 1)
    e = jnp.exp(th) + jnp.float32(_EPS)
    d = jnp.where(col == 0, th, e)
    y = d
    zero = jnp.zeros((1, _NB_KNOTS), jnp.float32)
    for s in (1, 2, 4, 8, 16, 32, 64):
        y = y + jnp.where(col >= s, jnp.roll(y, s, axis=1), zero)
    dy = jnp.concatenate([e[:, 1:], jnp.zeros((1, 1), jnp.float32)], axis=1)

    z = z_ref[...]
    zn = (z - jnp.float32(_X_MIN)) * jnp.float32(_SCALE)
    znc = jnp.minimum(jnp.maximum(zn, jnp.float32(0.0)),
                      jnp.float32(_NB_KNOTS - 2))
    ii = znc.astype(jnp.int32)
    t = zn - ii.astype(jnp.float32)
    y_b = jnp.broadcast_to(y, (_TC_BR, _NB_KNOTS))
    dy_b = jnp.broadcast_to(dy, (_TC_BR, _NB_KNOTS))
    yl = jnp.take_along_axis(y_b, ii, axis=1, mode="promise_in_bounds")
    dv = jnp.take_along_axis(dy_b, ii, axis=1, mode="promise_in_bounds")
    o_ref[...] = yl + t * dv


_spline_tc = pl.pallas_call(
    _spline_tc_body,
    grid=(_TC_ROWS // _TC_BR,),
    in_specs=[
        pl.BlockSpec((_TC_BR, _COLS), lambda i: (i + _SC_ROWS // _TC_BR, 0)),
        pl.BlockSpec((1, _NB_KNOTS), lambda i: (0, 0)),
    ],
    out_specs=pl.BlockSpec((_TC_BR, _COLS), lambda i: (i, 0)),
    out_shape=jax.ShapeDtypeStruct((_TC_ROWS, _COLS), jnp.float32),
)


def kernel(z, theta):
    sc_out = _spline_sc(z, theta)
    tc_out = _spline_tc(z, theta.reshape(1, _NB_KNOTS))
    return jnp.concatenate([sc_out, tc_out], axis=0)


# speedup vs baseline: 1.4343x; 1.3664x over previous
import functools

import jax
import jax.numpy as jnp
from jax import lax
from jax.experimental import pallas as pl
from jax.experimental.pallas import tpu as pltpu
from jax.experimental.pallas import tpu_sc as plsc

_NB_KNOTS = 128
_X_MIN = -3.0
_X_MAX = 3.0
_EPS = 1e-06
_SCALE = (_NB_KNOTS - 1) / (_X_MAX - _X_MIN)

_NC = 2
_NS = 16
_NW = _NC * _NS
_L = 16

_ROWS = 2048
_COLS = 4096
_SC_ROWS = 768
_TC_ROWS = _ROWS - _SC_ROWS
_TC_BR = 256

_RPW = _SC_ROWS // _NW
_CR = 8
_CC = _COLS // 2
_NG = _RPW // _CR


def _build_tables(theta_ref, y_ref, dy_ref):
    lane = lax.iota(jnp.int32, _L)
    zero = jnp.zeros((_L,), jnp.float32)
    carry = zero
    for k in range(_NB_KNOTS // _L):
        v = theta_ref[pl.ds(k * _L, _L)]
        d = jnp.exp(v) + jnp.float32(_EPS)
        if k == 0:
            d = jnp.where(lane == 0, v, d)
        c = d
        for s in (1, 2, 4, 8):
            y_ref[pl.ds(k * _L, _L)] = c
            shifted = plsc.load_gather(
                y_ref, [jnp.maximum(lane - s, 0) + k * _L])
            c = c + jnp.where(lane >= s, shifted, zero)
        c = c + carry
        y_ref[pl.ds(k * _L, _L)] = c
        carry = plsc.load_gather(
            y_ref, [jnp.full((_L,), k * _L + _L - 1, jnp.int32)]
        )
    for k in range(_NB_KNOTS // _L):
        idx = lane + k * _L
        yl = plsc.load_gather(y_ref, [idx])
        yr = plsc.load_gather(y_ref, [jnp.minimum(idx + 1, _NB_KNOTS - 1)])
        dy_ref[pl.ds(k * _L, _L)] = yr - yl


def _interp_block(inb, outb, y_ref, dy_ref):
    for r in range(_CR):
        @plsc.parallel_loop(0, _CC, step=_L, unroll=4)
        def body(off):
            zv = inb[r, pl.ds(off, _L)]
            zn = (zv - jnp.float32(_X_MIN)) * jnp.float32(_SCALE)
            znc = jnp.minimum(jnp.maximum(zn, jnp.float32(0.0)),
                              jnp.float32(_NB_KNOTS - 2))
            ii = znc.astype(jnp.int32)
            t = zn - ii.astype(jnp.float32)
            yl = plsc.load_gather(y_ref, [ii])
            dy = plsc.load_gather(dy_ref, [ii])
            outb[r, pl.ds(off, _L)] = yl + t * dy


@functools.partial(
    pl.kernel,
    mesh=plsc.VectorSubcoreMesh(core_axis_name="c", subcore_axis_name="s"),
    out_type=jax.ShapeDtypeStruct((_SC_ROWS, _COLS), jnp.float32),
    compiler_params=pltpu.CompilerParams(needs_layout_passes=False),
    scratch_types=[
        pltpu.VMEM((_NB_KNOTS,), jnp.float32),
        pltpu.VMEM((_NB_KNOTS,), jnp.float32),
        pltpu.VMEM((_NB_KNOTS,), jnp.float32),
        pltpu.VMEM((_CR, _CC), jnp.float32),
        pltpu.VMEM((_CR, _CC), jnp.float32),
        pltpu.VMEM((_CR, _CC), jnp.float32),
        pltpu.VMEM((_CR, _CC), jnp.float32),
        pltpu.SemaphoreType.DMA,
        pltpu.SemaphoreType.DMA,
        pltpu.SemaphoreType.DMA,
        pltpu.SemaphoreType.DMA,
    ],
)
def _spline_sc(z_hbm, theta_hbm, out_hbm,
               theta_v, y_v, dy_v, ib0, ib1, ob0, ob1, si0, si1, so0, so1):
    wid = lax.axis_index("s") * _NC + lax.axis_index("c")
    row0 = wid * _RPW

    pltpu.sync_copy(theta_hbm, theta_v)
    _build_tables(theta_v, y_v, dy_v)

    def in_slice(g, b):
        return z_hbm.at[pl.ds(row0 + g * _CR, _CR), pl.ds(b * _CC, _CC)]

    def out_slice(g, b):
        return out_hbm.at[pl.ds(row0 + g * _CR, _CR), pl.ds(b * _CC, _CC)]

    pltpu.async_copy(in_slice(0, 0), ib0, si0)
    pltpu.async_copy(in_slice(0, 1), ib1, si1)

    def group(g, carry):
        pairs = ((ib0, ob0, si0, so0), (ib1, ob1, si1, so1))
        for b, (inb, outb, si, so) in enumerate(pairs):
            pltpu.make_async_copy(in_slice(g, b), inb, si).wait()

            @pl.when(g > 0)
            def _():
                pltpu.make_async_copy(outb, out_slice(g, b), so).wait()

            _interp_block(inb, outb, y_v, dy_v)
            pltpu.async_copy(outb, out_slice(g, b), so)

            @pl.when(g + 1 < _NG)
            def _():
                pltpu.async_copy(in_slice(g + 1, b), inb, si)
        return carry

    lax.fori_loop(0, _NG, group, None)
    pltpu.make_async_copy(ob0, out_slice(_NG - 1, 0), so0).wait()
    pltpu.make_async_copy(ob1, out_slice(_NG - 1, 1), so1).wait()


def _spline_tc_body(z_ref, th_ref, o_ref):
    th = th_ref[...]
    col = lax.broadcasted_iota(jnp.int32, (1, _NB_KNOTS), 1)
    e = jnp.exp(th) + jnp.float32(_EPS)
    d = jnp.where(col == 0, th, e)
    y = d
    zero = jnp.zeros((1, _NB_KNOTS), jnp.float32)
    for s in (1, 2, 4, 8, 16, 32, 64):
        y = y + jnp.where(col >= s, jnp.roll(y, s, axis=1), zero)
    dy = jnp.concatenate([e[:, 1:], jnp.zeros((1, 1), jnp.float32)], axis=1)

    z = z_ref[...]
    zn = (z - jnp.float32(_X_MIN)) * jnp.float32(_SCALE)
    znc = jnp.minimum(jnp.maximum(zn, jnp.float32(0.0)),
                      jnp.float32(_NB_KNOTS - 2))
    ii = znc.astype(jnp.int32)
    t = zn - ii.astype(jnp.float32)
    y_b = jnp.broadcast_to(y, (_TC_BR, _NB_KNOTS))
    dy_b = jnp.broadcast_to(dy, (_TC_BR, _NB_KNOTS))
    yl = jnp.take_along_axis(y_b, ii, axis=1, mode="promise_in_bounds")
    dv = jnp.take_along_axis(dy_b, ii, axis=1, mode="promise_in_bounds")
    o_ref[...] = yl + t * dv


_spline_tc = pl.pallas_call(
    _spline_tc_body,
    grid=(_ROWS // _TC_BR,),
    in_specs=[
        pl.BlockSpec((_TC_BR, _COLS), lambda i: (i, 0)),
        pl.BlockSpec((1, _NB_KNOTS), lambda i: (0, 0)),
    ],
    out_specs=pl.BlockSpec((_TC_BR, _COLS), lambda i: (i, 0)),
    out_shape=jax.ShapeDtypeStruct((_ROWS, _COLS), jnp.float32),
)


def kernel(z, theta):
    return _spline_tc(z, theta.reshape(1, _NB_KNOTS))
